# baseline (speedup 1.0000x reference)
import functools

import jax
import jax.numpy as jnp
from jax.experimental import pallas as pl
from jax.experimental.pallas import tpu as pltpu


def _router_kernel(x_ref, wg_ref, logits_ref, probs_ref, *, n_experts):
    x = x_ref[...]
    wg = wg_ref[...]
    logits = jax.lax.dot_general(
        x, wg, (((1,), (1,)), ((), ())), preferred_element_type=jnp.float32)
    logits_ref[...] = logits
    lane = jax.lax.broadcasted_iota(jnp.int32, logits.shape, 1)
    masked = jnp.where(lane < n_experts, logits, -jnp.inf)
    m = jnp.max(masked, axis=1, keepdims=True)
    ex = jnp.where(lane < n_experts, jnp.exp(masked - m), 0.0)
    probs_ref[...] = ex / jnp.sum(ex, axis=1, keepdims=True)


def _ffn_kernel(te_ref, used_ref, sot_ref, w_ref, x_ref, w1_ref, w2_ref,
                out_ref, *, tile):
    del te_ref
    i = pl.program_id(0)

    @pl.when(used_ref[i] == 0)
    def _pad_tile():
        out_ref[...] = jnp.zeros_like(out_ref)

    @pl.when(used_ref[i] != 0)
    def _body():
        sot = sot_ref[...]
        n = sot.shape[1]
        slot_iota = jax.lax.broadcasted_iota(jnp.int32, (tile, n), 0) + i * tile
        onehot = slot_iota == sot
        g = onehot.astype(jnp.bfloat16)
        xg = jax.lax.dot_general(
            g, x_ref[...], (((1,), (0,)), ((), ())),
            preferred_element_type=jnp.float32)
        w1 = w1_ref[0]
        w2 = w2_ref[0]
        h = jax.lax.dot_general(
            xg.astype(jnp.bfloat16), w1, (((1,), (1,)), ((), ())),
            preferred_element_type=jnp.float32)
        h = 0.5 * h * (1.0 + jax.lax.erf(h * (2.0 ** -0.5)))
        o = jax.lax.dot_general(
            h.astype(jnp.bfloat16), w2, (((1,), (1,)), ((), ())),
            preferred_element_type=jnp.float32)
        wslot = jnp.max(jnp.where(onehot, w_ref[...], 0.0),
                        axis=1, keepdims=True)
        out_ref[...] = o * wslot


def _scatter_kernel(sot_ref, os_ref, out_ref, *, tile, n_slots):
    s_col = sot_ref[...][:, 0:1]
    iota = jax.lax.broadcasted_iota(jnp.int32, (tile, n_slots), 1)
    g = (iota == s_col).astype(jnp.float32)
    out_ref[...] = jax.lax.dot_general(
        g, os_ref[...], (((1,), (0,)), ((), ())),
        preferred_element_type=jnp.float32)


def kernel(x, Wg, W1, W2):
    B_, S_, H_ = x.shape
    E_, I_, _ = W1.shape
    N = B_ * S_
    x2 = x.reshape(N, H_)

    EP = 128
    wg_pad = jnp.zeros((EP, H_), jnp.float32).at[:E_].set(Wg)
    logits_pad, probs_pad = pl.pallas_call(
        functools.partial(_router_kernel, n_experts=E_),
        out_shape=(
            jax.ShapeDtypeStruct((N, EP), jnp.float32),
            jax.ShapeDtypeStruct((N, EP), jnp.float32),
        ),
    )(x2, wg_pad)
    logits = logits_pad[:, :E_]
    probs = probs_pad[:, :E_]

    cap = max(1, N // E_)
    aff = probs.T
    scores, idxs = jax.lax.top_k(aff, cap)
    sel = jnp.full((E_, N), -jnp.inf, probs.dtype)
    sel = sel.at[jnp.arange(E_)[:, None], idxs].set(scores)
    best_score = jnp.max(sel, axis=0)
    best_expert = jnp.argmax(sel, axis=0)
    assigned = best_score > -jnp.inf
    fallback = jnp.argmax(probs, axis=1)
    expert_idx = jnp.where(assigned, best_expert, fallback)
    weight = jnp.take_along_axis(probs, expert_idx[:, None], axis=1)[:, 0]

    T = 128
    num_tiles = N // T + E_
    S_slots = num_tiles * T
    oh = (expert_idx[:, None] == jnp.arange(E_)[None, :]).astype(jnp.int32)
    rank = jnp.take_along_axis(jnp.cumsum(oh, axis=0) - 1,
                               expert_idx[:, None], axis=1)[:, 0]
    c = jnp.sum(oh, axis=0)
    pc = ((c + T - 1) // T) * T
    pend = jnp.cumsum(pc)
    poff = pend - pc
    sot = (poff[expert_idx] + rank).astype(jnp.int32)
    tile_id = jnp.arange(num_tiles)
    tile_expert = jnp.minimum(
        jnp.searchsorted(pend, tile_id * T, side="right"),
        E_ - 1).astype(jnp.int32)
    tile_used = (tile_id * T < (poff + c)[tile_expert]).astype(jnp.int32)

    grid_spec = pltpu.PrefetchScalarGridSpec(
        num_scalar_prefetch=2,
        grid=(num_tiles,),
        in_specs=[
            pl.BlockSpec((1, N), lambda i, te, u: (0, 0)),
            pl.BlockSpec((1, N), lambda i, te, u: (0, 0)),
            pl.BlockSpec((N, H_), lambda i, te, u: (0, 0)),
            pl.BlockSpec((1, I_, H_), lambda i, te, u: (te[i], 0, 0)),
            pl.BlockSpec((1, H_, I_), lambda i, te, u: (te[i], 0, 0)),
        ],
        out_specs=pl.BlockSpec((T, H_), lambda i, te, u: (i, 0)),
    )
    o_slots = pl.pallas_call(
        functools.partial(_ffn_kernel, tile=T),
        grid_spec=grid_spec,
        out_shape=jax.ShapeDtypeStruct((S_slots, H_), jnp.float32),
    )(tile_expert, tile_used, sot[None, :], weight[None, :],
      x2.astype(jnp.bfloat16), W1.astype(jnp.bfloat16),
      W2.astype(jnp.bfloat16))

    sot_rep = jnp.broadcast_to(sot[:, None], (N, 128))
    out2 = pl.pallas_call(
        functools.partial(_scatter_kernel, tile=T, n_slots=S_slots),
        grid=(N // T,),
        in_specs=[
            pl.BlockSpec((T, 128), lambda i: (i, 0)),
            pl.BlockSpec((S_slots, H_), lambda i: (0, 0)),
        ],
        out_specs=pl.BlockSpec((T, H_), lambda i: (i, 0)),
        out_shape=jax.ShapeDtypeStruct((N, H_), jnp.float32),
    )(sot_rep, o_slots)

    return (out2.reshape(B_, S_, H_),
            weight.reshape(B_, S_),
            expert_idx.reshape(B_, S_),
            logits,
            probs)


# speedup vs baseline: 1.2465x; 1.2465x over previous
import functools

import jax
import jax.numpy as jnp
from jax.experimental import pallas as pl
from jax.experimental.pallas import tpu as pltpu


def _router_kernel(x_ref, wg_ref, logits_ref, probs_ref, *, n_experts):
    x = x_ref[...]
    wg = wg_ref[...]
    logits = jax.lax.dot_general(
        x, wg, (((1,), (1,)), ((), ())), preferred_element_type=jnp.float32)
    logits_ref[...] = logits
    lane = jax.lax.broadcasted_iota(jnp.int32, logits.shape, 1)
    masked = jnp.where(lane < n_experts, logits, -jnp.inf)
    m = jnp.max(masked, axis=1, keepdims=True)
    ex = jnp.where(lane < n_experts, jnp.exp(masked - m), 0.0)
    probs_ref[...] = ex / jnp.sum(ex, axis=1, keepdims=True)


def _ffn_kernel(te_ref, used_ref, sot_ref, w_ref, x_ref, w1_ref, w2_ref,
                out_ref, *, tile):
    del te_ref
    i = pl.program_id(0)

    @pl.when(used_ref[i] == 0)
    def _pad_tile():
        out_ref[...] = jnp.zeros_like(out_ref)

    @pl.when(used_ref[i] != 0)
    def _body():
        sot = sot_ref[...]
        n = sot.shape[1]
        slot_iota = jax.lax.broadcasted_iota(jnp.int32, (tile, n), 0) + i * tile
        onehot = slot_iota == sot
        g = onehot.astype(jnp.float32)
        xg = jax.lax.dot_general(
            g, x_ref[...], (((1,), (0,)), ((), ())),
            preferred_element_type=jnp.float32)
        w1 = w1_ref[0]
        w2 = w2_ref[0]
        h = jax.lax.dot_general(
            xg, w1, (((1,), (1,)), ((), ())),
            preferred_element_type=jnp.float32)
        h = 0.5 * h * (1.0 + jax.lax.erf(h * (2.0 ** -0.5)))
        o = jax.lax.dot_general(
            h, w2, (((1,), (1,)), ((), ())),
            preferred_element_type=jnp.float32)
        wslot = jnp.max(jnp.where(onehot, w_ref[...], 0.0),
                        axis=1, keepdims=True)
        out_ref[...] = o * wslot


def _scatter_kernel(sot_ref, os_ref, out_ref, *, tile, n_slots):
    s_col = sot_ref[...][:, 0:1]
    iota = jax.lax.broadcasted_iota(jnp.int32, (tile, n_slots), 1)
    g = (iota == s_col).astype(jnp.float32)
    out_ref[...] = jax.lax.dot_general(
        g, os_ref[...], (((1,), (0,)), ((), ())),
        preferred_element_type=jnp.float32)


def kernel(x, Wg, W1, W2):
    B_, S_, H_ = x.shape
    E_, I_, _ = W1.shape
    N = B_ * S_
    x2 = x.reshape(N, H_)

    EP = 128
    wg_pad = jnp.zeros((EP, H_), jnp.float32).at[:E_].set(Wg)
    logits_pad, probs_pad = pl.pallas_call(
        functools.partial(_router_kernel, n_experts=E_),
        out_shape=(
            jax.ShapeDtypeStruct((N, EP), jnp.float32),
            jax.ShapeDtypeStruct((N, EP), jnp.float32),
        ),
    )(x2, wg_pad)
    logits = logits_pad[:, :E_]
    probs = probs_pad[:, :E_]

    cap = max(1, N // E_)
    aff = probs.T
    scores, idxs = jax.lax.top_k(aff, cap)
    sel = jnp.full((E_, N), -jnp.inf, probs.dtype)
    sel = sel.at[jnp.arange(E_)[:, None], idxs].set(scores)
    best_score = jnp.max(sel, axis=0)
    best_expert = jnp.argmax(sel, axis=0)
    assigned = best_score > -jnp.inf
    fallback = jnp.argmax(probs, axis=1)
    expert_idx = jnp.where(assigned, best_expert, fallback)
    weight = jnp.take_along_axis(probs, expert_idx[:, None], axis=1)[:, 0]

    T = 128
    num_tiles = N // T + E_
    S_slots = num_tiles * T
    oh = (expert_idx[:, None] == jnp.arange(E_)[None, :]).astype(jnp.int32)
    rank = jnp.take_along_axis(jnp.cumsum(oh, axis=0) - 1,
                               expert_idx[:, None], axis=1)[:, 0]
    c = jnp.sum(oh, axis=0)
    pc = ((c + T - 1) // T) * T
    pend = jnp.cumsum(pc)
    poff = pend - pc
    sot = (poff[expert_idx] + rank).astype(jnp.int32)
    tile_id = jnp.arange(num_tiles)
    tile_expert = jnp.minimum(
        jnp.searchsorted(pend, tile_id * T, side="right"),
        E_ - 1).astype(jnp.int32)
    tile_used = (tile_id * T < (poff + c)[tile_expert]).astype(jnp.int32)

    grid_spec = pltpu.PrefetchScalarGridSpec(
        num_scalar_prefetch=2,
        grid=(num_tiles,),
        in_specs=[
            pl.BlockSpec((1, N), lambda i, te, u: (0, 0)),
            pl.BlockSpec((1, N), lambda i, te, u: (0, 0)),
            pl.BlockSpec((N, H_), lambda i, te, u: (0, 0)),
            pl.BlockSpec((1, I_, H_), lambda i, te, u: (te[i], 0, 0)),
            pl.BlockSpec((1, H_, I_), lambda i, te, u: (te[i], 0, 0)),
        ],
        out_specs=pl.BlockSpec((T, H_), lambda i, te, u: (i, 0)),
    )
    o_slots = pl.pallas_call(
        functools.partial(_ffn_kernel, tile=T),
        grid_spec=grid_spec,
        out_shape=jax.ShapeDtypeStruct((S_slots, H_), jnp.float32),
    )(tile_expert, tile_used, sot[None, :], weight[None, :], x2, W1, W2)

    sot_rep = jnp.broadcast_to(sot[:, None], (N, 128))
    out2 = pl.pallas_call(
        functools.partial(_scatter_kernel, tile=T, n_slots=S_slots),
        grid=(N // T,),
        in_specs=[
            pl.BlockSpec((T, 128), lambda i: (i, 0)),
            pl.BlockSpec((S_slots, H_), lambda i: (0, 0)),
        ],
        out_specs=pl.BlockSpec((T, H_), lambda i: (i, 0)),
        out_shape=jax.ShapeDtypeStruct((N, H_), jnp.float32),
    )(sot_rep, o_slots)

    return (out2.reshape(B_, S_, H_),
            weight.reshape(B_, S_),
            expert_idx.reshape(B_, S_),
            logits,
            probs)


# speedup vs baseline: 1.5424x; 1.2373x over previous
import functools

import jax
import jax.numpy as jnp
from jax.experimental import pallas as pl
from jax.experimental.pallas import tpu as pltpu


def _route_kernel(x_ref, wg_ref, logits_ref, probs_ref, sot_ref, w_ref,
                  ei_ref, tm_ref, *, n_experts, cap, tile, tm_rows):
    x = x_ref[...]
    wg = wg_ref[...]
    logits = jax.lax.dot_general(
        x, wg, (((1,), (1,)), ((), ())), preferred_element_type=jnp.float32)
    logits_ref[...] = logits
    n, ep = logits.shape
    lane = jax.lax.broadcasted_iota(jnp.int32, (n, ep), 1)
    elane = lane < n_experts
    masked = jnp.where(elane, logits, -jnp.inf)
    m = jnp.max(masked, axis=1, keepdims=True)
    ex = jnp.where(elane, jnp.exp(masked - m), 0.0)
    probs = ex / jnp.sum(ex, axis=1, keepdims=True)
    probs_ref[...] = probs

    bits = jax.lax.bitcast_convert_type(probs, jnp.int32)

    def bit_step(k, u):
        trial = u | jnp.left_shift(jnp.int32(1), 30 - k)
        cnt = jnp.sum((bits >= trial).astype(jnp.int32), axis=0, keepdims=True)
        return jnp.where(cnt >= cap, trial, u)

    thr = jax.lax.fori_loop(0, 31, bit_step, jnp.zeros((1, ep), jnp.int32))

    sel_mask = (bits >= thr) & elane
    sel = jnp.where(sel_mask, probs, -jnp.inf)
    bs = jnp.max(sel, axis=1, keepdims=True)
    be = jnp.min(jnp.where(sel == bs, lane, ep), axis=1, keepdims=True)
    fs = jnp.max(masked_probs := jnp.where(elane, probs, -jnp.inf), axis=1,
                 keepdims=True)
    fe = jnp.min(jnp.where((masked_probs == fs), lane, ep), axis=1,
                 keepdims=True)
    assigned = bs > -jnp.inf
    ei = jnp.where(assigned, be, fe)
    wgt = jnp.where(assigned, bs, fs)

    oh = (lane == ei).astype(jnp.float32)
    r_i = jax.lax.broadcasted_iota(jnp.int32, (tile, tile), 0)
    c_i = jax.lax.broadcasted_iota(jnp.int32, (tile, tile), 1)
    ltri = (r_i > c_i).astype(jnp.float32)
    carry = jnp.zeros((1, ep), jnp.float32)
    rank_blocks = []
    for b in range(n // tile):
        ohb = jax.lax.slice(oh, (b * tile, 0), ((b + 1) * tile, ep))
        pref = jax.lax.dot_general(
            ltri, ohb, (((1,), (0,)), ((), ())),
            preferred_element_type=jnp.float32) + carry
        rank_blocks.append(jnp.sum(pref * ohb, axis=1, keepdims=True))
        carry = carry + jnp.sum(ohb, axis=0, keepdims=True)
    rank = jnp.concatenate(rank_blocks, axis=0)
    c = carry
    pcf = jnp.ceil(c / tile) * tile
    utri = (r_i[:ep, :ep] < c_i[:ep, :ep]).astype(jnp.float32)
    poff = jax.lax.dot_general(
        pcf, utri, (((1,), (0,)), ((), ())),
        preferred_element_type=jnp.float32)
    sot = (jnp.sum(oh * poff, axis=1, keepdims=True) + rank).astype(jnp.int32)
    sot_ref[...] = jnp.broadcast_to(sot, (n, ep))
    w_ref[...] = jnp.broadcast_to(wgt, (n, ep))
    ei_ref[...] = jnp.broadcast_to(ei, (n, ep))

    pend = poff + pcf
    it_f = (jax.lax.broadcasted_iota(jnp.int32, (tm_rows, ep), 0)
            * tile).astype(jnp.float32)
    tlane = jax.lax.broadcasted_iota(jnp.int32, (tm_rows, ep), 1)
    telane = tlane < n_experts
    te = jnp.minimum(
        jnp.sum(jnp.where((pend <= it_f) & telane, 1, 0), axis=1,
                keepdims=True),
        n_experts - 1)
    lim = jnp.sum(jnp.where(tlane == te, poff + c, 0.0), axis=1, keepdims=True)
    used = (it_f[:, 0:1] < lim).astype(jnp.int32)
    tm_ref[...] = jnp.where(tlane == 0, te,
                            jnp.where(tlane == 1, used, 0))


def _ffn_kernel(te_ref, used_ref, sot_ref, w_ref, x_ref, w1_ref, w2_ref,
                out_ref, *, tile):
    del te_ref
    i = pl.program_id(0)

    @pl.when(used_ref[i] == 0)
    def _pad_tile():
        out_ref[...] = jnp.zeros_like(out_ref)

    @pl.when(used_ref[i] != 0)
    def _body():
        sot = sot_ref[...]
        n = sot.shape[1]
        slot_iota = jax.lax.broadcasted_iota(jnp.int32, (tile, n), 0) + i * tile
        onehot = slot_iota == sot
        g = onehot.astype(jnp.float32)
        xg = jax.lax.dot_general(
            g, x_ref[...], (((1,), (0,)), ((), ())),
            preferred_element_type=jnp.float32)
        w1 = w1_ref[0]
        w2 = w2_ref[0]
        h = jax.lax.dot_general(
            xg, w1, (((1,), (1,)), ((), ())),
            preferred_element_type=jnp.float32)
        h = 0.5 * h * (1.0 + jax.lax.erf(h * (2.0 ** -0.5)))
        o = jax.lax.dot_general(
            h, w2, (((1,), (1,)), ((), ())),
            preferred_element_type=jnp.float32)
        wslot = jnp.max(jnp.where(onehot, w_ref[...], 0.0),
                        axis=1, keepdims=True)
        out_ref[...] = o * wslot


def _scatter_kernel(sot_ref, os_ref, out_ref, *, tile, n_slots):
    s_col = sot_ref[...][:, 0:1]
    iota = jax.lax.broadcasted_iota(jnp.int32, (tile, n_slots), 1)
    g = (iota == s_col).astype(jnp.float32)
    out_ref[...] = jax.lax.dot_general(
        g, os_ref[...], (((1,), (0,)), ((), ())),
        preferred_element_type=jnp.float32)


def kernel(x, Wg, W1, W2):
    B_, S_, H_ = x.shape
    E_, I_, _ = W1.shape
    N = B_ * S_
    x2 = x.reshape(N, H_)

    T = 128
    EP = 128
    cap = max(1, N // E_)
    num_tiles = N // T + E_
    S_slots = num_tiles * T
    TM = ((num_tiles + 7) // 8) * 8
    wg_pad = jnp.zeros((EP, H_), jnp.float32).at[:E_].set(Wg)
    logits_pad, probs_pad, sot_rep, w_rep, ei_rep, tmeta = pl.pallas_call(
        functools.partial(_route_kernel, n_experts=E_, cap=cap, tile=T,
                          tm_rows=TM),
        out_shape=(
            jax.ShapeDtypeStruct((N, EP), jnp.float32),
            jax.ShapeDtypeStruct((N, EP), jnp.float32),
            jax.ShapeDtypeStruct((N, EP), jnp.int32),
            jax.ShapeDtypeStruct((N, EP), jnp.float32),
            jax.ShapeDtypeStruct((N, EP), jnp.int32),
            jax.ShapeDtypeStruct((TM, EP), jnp.int32),
        ),
    )(x2, wg_pad)
    logits = logits_pad[:, :E_]
    probs = probs_pad[:, :E_]
    weight = w_rep[:, 0]
    expert_idx = ei_rep[:, 0]
    sot_row = sot_rep[:, 0].reshape(1, N)
    tile_expert = tmeta[:num_tiles, 0]
    tile_used = tmeta[:num_tiles, 1]

    grid_spec = pltpu.PrefetchScalarGridSpec(
        num_scalar_prefetch=2,
        grid=(num_tiles,),
        in_specs=[
            pl.BlockSpec((1, N), lambda i, te, u: (0, 0)),
            pl.BlockSpec((1, N), lambda i, te, u: (0, 0)),
            pl.BlockSpec((N, H_), lambda i, te, u: (0, 0)),
            pl.BlockSpec((1, I_, H_), lambda i, te, u: (te[i], 0, 0)),
            pl.BlockSpec((1, H_, I_), lambda i, te, u: (te[i], 0, 0)),
        ],
        out_specs=pl.BlockSpec((T, H_), lambda i, te, u: (i, 0)),
    )
    o_slots = pl.pallas_call(
        functools.partial(_ffn_kernel, tile=T),
        grid_spec=grid_spec,
        out_shape=jax.ShapeDtypeStruct((S_slots, H_), jnp.float32),
    )(tile_expert, tile_used, sot_row, weight.reshape(1, N), x2, W1, W2)

    out2 = pl.pallas_call(
        functools.partial(_scatter_kernel, tile=T, n_slots=S_slots),
        grid=(N // T,),
        in_specs=[
            pl.BlockSpec((T, 128), lambda i: (i, 0)),
            pl.BlockSpec((S_slots, H_), lambda i: (0, 0)),
        ],
        out_specs=pl.BlockSpec((T, H_), lambda i: (i, 0)),
        out_shape=jax.ShapeDtypeStruct((N, H_), jnp.float32),
    )(sot_rep, o_slots)

    return (out2.reshape(B_, S_, H_),
            weight.reshape(B_, S_),
            expert_idx.reshape(B_, S_),
            logits,
            probs)
